# baseline (speedup 1.0000x reference)
import functools

import jax
import jax.numpy as jnp
from jax import lax
from jax.experimental import pallas as pl
from jax.experimental.pallas import tpu as pltpu
from jax.experimental.pallas import tpu_sc as plsc

_T, _D, _DO, _E = 4096, 1024, 1024, 8
_BT = 256
_NT = _T // _BT
_NTILES = 2 * _T // _BT + _E
_SROWS = _NTILES * _BT
_NW = 32
_TPW = _T // _NW
_CH = 64


def _iota_e(rows):
    return lax.broadcasted_iota(jnp.int32, (rows, _E), 1)


def _router_weights(x, wr, br):
    logits = jnp.dot(x, wr, preferred_element_type=jnp.float32) + br
    m = jnp.max(logits, axis=-1, keepdims=True)
    p = jnp.exp(logits - m)
    w = p / jnp.sum(p, axis=-1, keepdims=True)
    rank = jnp.zeros_like(w)
    for j in range(_E):
        rank = rank + (w[:, j : j + 1] > w).astype(jnp.float32)
    return jnp.where(rank < 2.0, w, 0.0)


def _sides(sw):
    mask = sw > 0.0
    ie = _iota_e(sw.shape[0])
    e_lo = jnp.min(jnp.where(mask, ie, _E), axis=1, keepdims=True)
    e_hi = jnp.max(jnp.where(mask, ie, -1), axis=1, keepdims=True)
    w_lo = jnp.sum(jnp.where(ie == e_lo, sw, 0.0), axis=1, keepdims=True)
    w_hi = jnp.sum(jnp.where(ie == e_hi, sw, 0.0), axis=1, keepdims=True)
    return e_lo, e_hi, w_lo, w_hi


def _a_body(x_ref, wr_ref, br_ref,
            sw_ref, elo_ref, ehi_ref, plo_ref, phi_ref,
            gs_ref, texp_ref, cnt_ref):
    i = pl.program_id(0)

    @pl.when(i == 0)
    def _init():
        cnt_ref[...] = jnp.zeros((1, _E), jnp.float32)

    x = x_ref[...]
    sw = _router_weights(x, wr_ref[...], br_ref[...])
    sw_ref[...] = sw
    mask = (sw > 0.0).astype(jnp.float32)
    r = lax.broadcasted_iota(jnp.int32, (_BT, _BT), 0)
    c = lax.broadcasted_iota(jnp.int32, (_BT, _BT), 1)
    tri = (r > c).astype(jnp.bfloat16)
    rank_local = jnp.dot(tri, mask.astype(jnp.bfloat16),
                         preferred_element_type=jnp.float32)
    cnt = cnt_ref[...]
    posrel = cnt + rank_local
    e_lo, e_hi, _, _ = _sides(sw)
    ie = _iota_e(_BT)
    plo = jnp.sum(jnp.where(ie == e_lo, posrel, 0.0), axis=1, keepdims=True)
    phi = jnp.sum(jnp.where(ie == e_hi, posrel, 0.0), axis=1, keepdims=True)
    elo_ref[...] = e_lo
    ehi_ref[...] = e_hi
    plo_ref[...] = plo.astype(jnp.int32)
    phi_ref[...] = phi.astype(jnp.int32)
    cnt_new = cnt + jnp.sum(mask, axis=0, keepdims=True)
    cnt_ref[...] = cnt_new

    @pl.when(i == _NT - 1)
    def _finalize():
        ptiles = jnp.floor((cnt_new + (_BT - 1.0)) * (1.0 / _BT))
        re = lax.broadcasted_iota(jnp.int32, (_E, _E), 0)
        ce = lax.broadcasted_iota(jnp.int32, (_E, _E), 1)
        ex = (re < ce).astype(jnp.bfloat16)
        tstart = jnp.dot(ptiles.astype(jnp.bfloat16), ex,
                         preferred_element_type=jnp.float32)
        gs_ref[...] = (tstart * float(_BT)).astype(jnp.int32)
        texp = jnp.zeros((1, 64), jnp.float32)
        pp = lax.broadcasted_iota(jnp.int32, (1, 64), 1).astype(jnp.float32)
        for e in range(_E):
            ts_e = lax.slice(tstart, (0, e), (1, e + 1))
            texp = texp + (pp >= ts_e).astype(jnp.float32)
        texp_ref[...] = (texp - 1.0).astype(jnp.int32)


def _a2_body(plo_ref, phi_ref, elo_ref, ehi_ref, gs_ref, alo_ref, ahi_ref):
    gs = gs_ref[...].astype(jnp.float32)
    for (p_ref, e_ref, o_ref) in ((plo_ref, elo_ref, alo_ref),
                                  (phi_ref, ehi_ref, ahi_ref)):
        p = p_ref[...].astype(jnp.float32)
        e = e_ref[...]
        out = p
        for ex in range(_E):
            g = lax.slice(gs, (0, ex), (0 + 1, ex + 1))
            out = out + jnp.where(e == ex, g, 0.0)
        o_ref[...] = out.astype(jnp.int32)


def _d_body(texp_ref, xs_ref, web_ref, ys_ref):
    del texp_ref
    ys_ref[...] = jnp.dot(xs_ref[...].astype(jnp.bfloat16), web_ref[0],
                          preferred_element_type=jnp.float32)


def _e_body(y1_ref, y2_ref, sw_ref, be_ref, o_ref):
    sw = sw_ref[...]
    _, _, w_lo, w_hi = _sides(sw)
    acc = jnp.dot(sw, be_ref[...], preferred_element_type=jnp.float32)
    acc = acc + w_lo * y1_ref[...].astype(jnp.float32)
    acc = acc + w_hi * y2_ref[...].astype(jnp.float32)
    o_ref[...] = acc


_vector_mesh = None


def _get_mesh():
    global _vector_mesh
    if _vector_mesh is None:
        _vector_mesh = plsc.VectorSubcoreMesh(core_axis_name="c",
                                              subcore_axis_name="s")
    return _vector_mesh


def _sc_scatter(xb, plo, phi):
    @functools.partial(
        pl.kernel, mesh=_get_mesh(),
        out_type=jax.ShapeDtypeStruct((_SROWS, _D), jnp.float32),
        scratch_types=[
            pltpu.VMEM((_CH,), jnp.int32),
            pltpu.VMEM((_CH,), jnp.int32),
            pltpu.VMEM((_CH, _D), jnp.float32),
            pltpu.SemaphoreType.DMA,
        ],
    )
    def k(xb_hbm, plo_hbm, phi_hbm, xs_hbm, ilo_v, ihi_v, rows_v, sem):
        wid = lax.axis_index("s") * 2 + lax.axis_index("c")
        for c in range(_TPW // _CH):
            base = wid * _TPW + c * _CH
            pltpu.sync_copy(plo_hbm.at[pl.ds(base, _CH)], ilo_v)
            pltpu.sync_copy(phi_hbm.at[pl.ds(base, _CH)], ihi_v)
            pltpu.sync_copy(xb_hbm.at[pl.ds(base, _CH)], rows_v)
            cp1 = pltpu.async_copy(rows_v, xs_hbm.at[ilo_v], sem)
            cp2 = pltpu.async_copy(rows_v, xs_hbm.at[ihi_v], sem)
            cp1.wait()
            cp2.wait()

    return k(xb, plo, phi)


def _sc_gather(ys, plo, phi):
    @functools.partial(
        pl.kernel, mesh=_get_mesh(),
        out_type=[jax.ShapeDtypeStruct((_T, _DO), jnp.float32),
                  jax.ShapeDtypeStruct((_T, _DO), jnp.float32)],
        scratch_types=[
            pltpu.VMEM((_CH,), jnp.int32),
            pltpu.VMEM((_CH, _DO), jnp.float32),
            pltpu.SemaphoreType.DMA,
        ],
    )
    def k(ys_hbm, plo_hbm, phi_hbm, y1_hbm, y2_hbm, idx_v, rows_v, sem):
        wid = lax.axis_index("s") * 2 + lax.axis_index("c")
        for c in range(_TPW // _CH):
            base = wid * _TPW + c * _CH
            for (p_hbm, o_hbm) in ((plo_hbm, y1_hbm), (phi_hbm, y2_hbm)):
                pltpu.sync_copy(p_hbm.at[pl.ds(base, _CH)], idx_v)
                pltpu.async_copy(ys_hbm.at[idx_v], rows_v, sem).wait()
                pltpu.sync_copy(rows_v, o_hbm.at[pl.ds(base, _CH)])

    return k(ys, plo, phi)


def kernel(x, Wr, br, We, be):
    br2 = br.reshape(1, _E)
    web = We.astype(jnp.bfloat16)

    sw, elo, ehi, plo_r, phi_r, gs, texp = pl.pallas_call(
        _a_body,
        grid=(_NT,),
        in_specs=[
            pl.BlockSpec((_BT, _D), lambda i: (i, 0)),
            pl.BlockSpec((_D, _E), lambda i: (0, 0)),
            pl.BlockSpec((1, _E), lambda i: (0, 0)),
        ],
        out_specs=[
            pl.BlockSpec((_BT, _E), lambda i: (i, 0)),
            pl.BlockSpec((_BT, 1), lambda i: (i, 0)),
            pl.BlockSpec((_BT, 1), lambda i: (i, 0)),
            pl.BlockSpec((_BT, 1), lambda i: (i, 0)),
            pl.BlockSpec((_BT, 1), lambda i: (i, 0)),
            pl.BlockSpec((1, _E), lambda i: (0, 0)),
            pl.BlockSpec((1, 64), lambda i: (0, 0)),
        ],
        out_shape=[
            jax.ShapeDtypeStruct((_T, _E), jnp.float32),
            jax.ShapeDtypeStruct((_T, 1), jnp.int32),
            jax.ShapeDtypeStruct((_T, 1), jnp.int32),
            jax.ShapeDtypeStruct((_T, 1), jnp.int32),
            jax.ShapeDtypeStruct((_T, 1), jnp.int32),
            jax.ShapeDtypeStruct((1, _E), jnp.int32),
            jax.ShapeDtypeStruct((1, 64), jnp.int32),
        ],
        scratch_shapes=[pltpu.VMEM((1, _E), jnp.float32)],
        compiler_params=pltpu.CompilerParams(
            dimension_semantics=("arbitrary",),
        ),
    )(x, Wr, br2)

    plo, phi = pl.pallas_call(
        _a2_body,
        grid=(_NT,),
        in_specs=[
            pl.BlockSpec((_BT, 1), lambda i: (i, 0)),
            pl.BlockSpec((_BT, 1), lambda i: (i, 0)),
            pl.BlockSpec((_BT, 1), lambda i: (i, 0)),
            pl.BlockSpec((_BT, 1), lambda i: (i, 0)),
            pl.BlockSpec((1, _E), lambda i: (0, 0)),
        ],
        out_specs=[
            pl.BlockSpec((_BT, 1), lambda i: (i, 0)),
            pl.BlockSpec((_BT, 1), lambda i: (i, 0)),
        ],
        out_shape=[
            jax.ShapeDtypeStruct((_T, 1), jnp.int32),
            jax.ShapeDtypeStruct((_T, 1), jnp.int32),
        ],
    )(plo_r, phi_r, elo, ehi, gs)

    plo_f = plo.reshape(_T)
    phi_f = phi.reshape(_T)
    xs = _sc_scatter(x, plo_f, phi_f)

    ys = pl.pallas_call(
        _d_body,
        grid_spec=pltpu.PrefetchScalarGridSpec(
            num_scalar_prefetch=1,
            grid=(_NTILES,),
            in_specs=[
                pl.BlockSpec((_BT, _D), lambda i, texp_ref: (i, 0)),
                pl.BlockSpec((1, _D, _DO),
                             lambda i, texp_ref: (texp_ref[i], 0, 0)),
            ],
            out_specs=pl.BlockSpec((_BT, _DO), lambda i, texp_ref: (i, 0)),
        ),
        out_shape=jax.ShapeDtypeStruct((_SROWS, _DO), jnp.float32),
        compiler_params=pltpu.CompilerParams(
            dimension_semantics=("arbitrary",),
        ),
    )(texp.reshape(64), xs, web)

    y1, y2 = _sc_gather(ys, plo_f, phi_f)

    return pl.pallas_call(
        _e_body,
        grid=(_NT,),
        in_specs=[
            pl.BlockSpec((_BT, _DO), lambda i: (i, 0)),
            pl.BlockSpec((_BT, _DO), lambda i: (i, 0)),
            pl.BlockSpec((_BT, _E), lambda i: (i, 0)),
            pl.BlockSpec((_E, _DO), lambda i: (0, 0)),
        ],
        out_specs=pl.BlockSpec((_BT, _DO), lambda i: (i, 0)),
        out_shape=jax.ShapeDtypeStruct((_T, _DO), jnp.float32),
    )(y1, y2, sw, be)


# speedup vs baseline: 1.8863x; 1.8863x over previous
import jax
import jax.numpy as jnp
from jax.experimental import pallas as pl
from jax.experimental.pallas import tpu as pltpu

_T, _D, _DO, _E = 4096, 1024, 1024, 8
_BT = 1024


def _moe_body(x_ref, wr_ref, br_ref, web_ref, be_ref, o_ref):
    x = x_ref[...]
    logits = jnp.dot(x, wr_ref[...], preferred_element_type=jnp.float32)
    logits = logits + br_ref[...]
    m = jnp.max(logits, axis=-1, keepdims=True)
    p = jnp.exp(logits - m)
    w = p / jnp.sum(p, axis=-1, keepdims=True)
    rank = jnp.zeros_like(w)
    for j in range(_E):
        rank = rank + (w[:, j : j + 1] > w).astype(jnp.float32)
    sw = jnp.where(rank < 2.0, w, 0.0)
    acc = jnp.dot(sw, be_ref[...], preferred_element_type=jnp.float32)
    xb = x.astype(jnp.bfloat16)
    for e in range(_E):
        y = jnp.dot(xb, web_ref[e], preferred_element_type=jnp.float32)
        acc = acc + sw[:, e : e + 1] * y
    o_ref[...] = acc


def kernel(x, Wr, br, We, be):
    br2 = br.reshape(1, _E)
    web = We.astype(jnp.bfloat16)
    return pl.pallas_call(
        _moe_body,
        grid=(_T // _BT,),
        in_specs=[
            pl.BlockSpec((_BT, _D), lambda i: (i, 0)),
            pl.BlockSpec((_D, _E), lambda i: (0, 0)),
            pl.BlockSpec((1, _E), lambda i: (0, 0)),
            pl.BlockSpec((_E, _D, _DO), lambda i: (0, 0, 0)),
            pl.BlockSpec((_E, _DO), lambda i: (0, 0)),
        ],
        out_specs=pl.BlockSpec((_BT, _DO), lambda i: (i, 0)),
        out_shape=jax.ShapeDtypeStruct((_T, _DO), jnp.float32),
        compiler_params=pltpu.CompilerParams(
            dimension_semantics=("arbitrary",),
        ),
    )(x, Wr, br2, web, be)
